# Initial kernel scaffold; baseline (speedup 1.0000x reference)
import jax
import jax.numpy as jnp
from jax.experimental import pallas as pl


def kernel(h_dag, obs_ptr, W1, b1, W2, b2, W3, b3):
    raise NotImplementedError("write your pallas kernel here")


# speedup vs baseline: 11.2510x; 11.2510x over previous
import jax
import jax.numpy as jnp
from jax.experimental import pallas as pl
from jax.experimental.pallas import tpu as pltpu

N_TOK = 32768
DIM = 128
NSEG = 16
TILE = 4096
GRID = N_TOK // TILE


def _fused_kernel(x_ref, lo_ref, hi_ref, w1_ref, b1_ref, w2_ref, b2_ref,
                  w3_ref, b3_ref, out_ref, acc_ref, cnt_ref):
    pid = pl.program_id(0)

    @pl.when(pid == 0)
    def _init():
        acc_ref[...] = jnp.zeros_like(acc_ref)
        cnt_ref[...] = jnp.zeros_like(cnt_ref)

    x = x_ref[...]
    h1 = jnp.dot(x, w1_ref[...], preferred_element_type=jnp.float32) + b1_ref[...]
    h1 = jnp.where(h1 >= 0, h1, 0.2 * h1)
    h2 = jnp.dot(h1, w2_ref[...], preferred_element_type=jnp.float32) + b2_ref[...]
    h2 = jnp.where(h2 >= 0, h2, 0.2 * h2)

    cols = jax.lax.broadcasted_iota(jnp.int32, (NSEG, TILE), 1) + pid * TILE
    m = jnp.logical_and(cols >= lo_ref[...], cols < hi_ref[...]).astype(jnp.float32)

    acc_ref[...] += jnp.dot(m, h2, preferred_element_type=jnp.float32)
    cnt_ref[...] += jnp.sum(m, axis=1, keepdims=True)

    @pl.when(pid == GRID - 1)
    def _finish():
        out_ref[...] = (
            jnp.dot(acc_ref[...], w3_ref[...], preferred_element_type=jnp.float32)
            + cnt_ref[...] * b3_ref[...]
        )


def kernel(h_dag, obs_ptr, W1, b1, W2, b2, W3, b3):
    lo = obs_ptr[:-1].astype(jnp.int32).reshape(NSEG, 1)
    hi = obs_ptr[1:].astype(jnp.int32).reshape(NSEG, 1)

    const = lambda *_: (0, 0)
    out = pl.pallas_call(
        _fused_kernel,
        grid=(GRID,),
        in_specs=[
            pl.BlockSpec((TILE, DIM), lambda i: (i, 0)),
            pl.BlockSpec((NSEG, 1), const),
            pl.BlockSpec((NSEG, 1), const),
            pl.BlockSpec((DIM, 32), const),
            pl.BlockSpec((1, 32), const),
            pl.BlockSpec((32, 16), const),
            pl.BlockSpec((1, 16), const),
            pl.BlockSpec((16, DIM), const),
            pl.BlockSpec((1, DIM), const),
        ],
        out_specs=pl.BlockSpec((NSEG, DIM), const),
        out_shape=jax.ShapeDtypeStruct((NSEG, DIM), jnp.float32),
        scratch_shapes=[
            pltpu.VMEM((NSEG, 16), jnp.float32),
            pltpu.VMEM((NSEG, 1), jnp.float32),
        ],
        compiler_params=pltpu.CompilerParams(
            dimension_semantics=("arbitrary",),
        ),
    )(h_dag, lo, hi, W1, b1.reshape(1, 32), W2, b2.reshape(1, 16),
      W3, b3.reshape(1, DIM))
    return out


# speedup vs baseline: 12.5189x; 1.1127x over previous
import jax
import jax.numpy as jnp
from jax.experimental import pallas as pl
from jax.experimental.pallas import tpu as pltpu

N_TOK = 32768
DIM = 128
NSEG = 16
TILE = 8192
GRID = N_TOK // TILE


def _fused_kernel(x_ref, lo_ref, hi_ref, w1_ref, b1_ref, w2_ref, b2_ref,
                  w3_ref, b3_ref, out_ref, acc_ref, cnt_ref):
    pid = pl.program_id(0)

    @pl.when(pid == 0)
    def _init():
        acc_ref[...] = jnp.zeros_like(acc_ref)
        cnt_ref[...] = jnp.zeros_like(cnt_ref)

    x = x_ref[...]
    h1 = jnp.dot(x, w1_ref[...], preferred_element_type=jnp.float32) + b1_ref[...]
    h1 = jnp.where(h1 >= 0, h1, 0.2 * h1)
    h2 = jnp.dot(h1, w2_ref[...], preferred_element_type=jnp.float32) + b2_ref[...]
    h2 = jnp.where(h2 >= 0, h2, 0.2 * h2)

    cols = jax.lax.broadcasted_iota(jnp.int32, (NSEG, TILE), 1) + pid * TILE
    m = jnp.logical_and(cols >= lo_ref[...], cols < hi_ref[...]).astype(jnp.float32)

    acc_ref[...] += jnp.dot(m, h2, preferred_element_type=jnp.float32)
    cnt_ref[...] += jnp.sum(m, axis=1, keepdims=True)

    @pl.when(pid == GRID - 1)
    def _finish():
        out_ref[...] = (
            jnp.dot(acc_ref[...], w3_ref[...], preferred_element_type=jnp.float32)
            + cnt_ref[...] * b3_ref[...]
        )


def kernel(h_dag, obs_ptr, W1, b1, W2, b2, W3, b3):
    lo = obs_ptr[:-1].astype(jnp.int32).reshape(NSEG, 1)
    hi = obs_ptr[1:].astype(jnp.int32).reshape(NSEG, 1)

    const = lambda *_: (0, 0)
    out = pl.pallas_call(
        _fused_kernel,
        grid=(GRID,),
        in_specs=[
            pl.BlockSpec((TILE, DIM), lambda i: (i, 0)),
            pl.BlockSpec((NSEG, 1), const),
            pl.BlockSpec((NSEG, 1), const),
            pl.BlockSpec((DIM, 32), const),
            pl.BlockSpec((1, 32), const),
            pl.BlockSpec((32, 16), const),
            pl.BlockSpec((1, 16), const),
            pl.BlockSpec((16, DIM), const),
            pl.BlockSpec((1, DIM), const),
        ],
        out_specs=pl.BlockSpec((NSEG, DIM), const),
        out_shape=jax.ShapeDtypeStruct((NSEG, DIM), jnp.float32),
        scratch_shapes=[
            pltpu.VMEM((NSEG, 16), jnp.float32),
            pltpu.VMEM((NSEG, 1), jnp.float32),
        ],
        compiler_params=pltpu.CompilerParams(
            dimension_semantics=("arbitrary",),
        ),
    )(h_dag, lo, hi, W1, b1.reshape(1, 32), W2, b2.reshape(1, 16),
      W3, b3.reshape(1, DIM))
    return out
